# baseline (speedup 1.0000x reference)
import functools

import jax
import jax.numpy as jnp
from jax import lax
from jax.experimental import pallas as pl
from jax.experimental.pallas import tpu as pltpu
from jax.experimental.pallas import tpu_sc as plsc

B = 4096
F = 26
V = 100000
D = 16

NC = 2
NS = 16
L = 16
NW = NC * NS
BPW = B // NW
NG = BPW // L


@functools.cache
def _build_fm_sc():
    mesh = plsc.VectorSubcoreMesh(
        core_axis_name="c", subcore_axis_name="s", num_cores=NC, num_subcores=NS
    )

    @functools.partial(
        pl.kernel,
        out_type=jax.ShapeDtypeStruct((B,), jnp.float32),
        mesh=mesh,
        scratch_types=[
            pltpu.VMEM((F, BPW), jnp.int32),
            pltpu.VMEM((F, BPW, D), jnp.float32),
            pltpu.VMEM((F, BPW), jnp.float32),
            pltpu.VMEM((BPW * D,), jnp.float32),
            pltpu.VMEM((BPW,), jnp.float32),
            pltpu.SemaphoreType.DMA,
        ],
        compiler_params=pltpu.CompilerParams(
            needs_layout_passes=False, use_tc_tiling_on_sc=False
        ),
    )
    def _fm_sc(idx_hbm, emb1_hbm, emb2_hbm, out_hbm,
               idx_v, rows_v, first_v, p2_v, out_v, sem):
        w = lax.axis_index("c") * NS + lax.axis_index("s")

        pltpu.sync_copy(idx_hbm.at[w], idx_v)

        def _issue(f, carry):
            pltpu.make_async_copy(emb2_hbm.at[idx_v.at[f]], rows_v.at[f], sem).start()
            pltpu.make_async_copy(emb1_hbm.at[idx_v.at[f]], first_v.at[f], sem).start()
            return carry

        lax.fori_loop(0, F, _issue, 0)

        def _drain(f, carry):
            pltpu.make_async_copy(emb2_hbm.at[idx_v.at[f]], rows_v.at[f], sem).wait()
            pltpu.make_async_copy(emb1_hbm.at[idx_v.at[f]], first_v.at[f], sem).wait()
            return carry

        lax.fori_loop(0, F, _drain, 0)

        def _sample(i, carry):
            acc = rows_v[0, i, :]
            acc2 = acc * acc
            for f in range(1, F):
                r = rows_v[f, i, :]
                acc = acc + r
                acc2 = acc2 + r * r
            p2_v[pl.ds(i * D, D)] = (acc * acc - acc2) * 0.5
            return carry

        lax.fori_loop(0, BPW, _sample, 0)

        lane = lax.iota(jnp.int32, L)
        for g in range(NG):
            t = first_v[0, pl.ds(g * L, L)]
            for f in range(1, F):
                t = t + first_v[f, pl.ds(g * L, L)]
            base_idx = lane * D + g * L * D
            for d in range(D):
                t = t + plsc.load_gather(p2_v, [base_idx + d])
            out_v[pl.ds(g * L, L)] = t

        pltpu.sync_copy(out_v, out_hbm.at[pl.ds(w * BPW, BPW)])

    return _fm_sc


def kernel(sparse_idx, emb_first, emb_second):
    offsets = (jnp.arange(F, dtype=sparse_idx.dtype) * V)[None, :]
    flat_idx = sparse_idx + offsets
    idx = flat_idx.reshape(NW, BPW, F).transpose(0, 2, 1)
    out = _build_fm_sc()(idx, emb_first.reshape(-1), emb_second)
    return out[:, None]
